# baseline (speedup 1.0000x reference)
import functools

import jax
import jax.numpy as jnp
from jax import lax
from jax.experimental import pallas as pl
from jax.experimental.pallas import tpu as pltpu
from jax.experimental.pallas import tpu_sc as plsc

VOCAB_ROWS = 1000000
EMBED = 64
OUT_DIM = 128
BATCH = 4096
HIST = 200

NC = 2
NS = 16
NW = NC * NS
SPT = BATCH // NW
C0, C1 = 104, 96
VR = EMBED // 16
NBUF = 3

_mesh = plsc.VectorSubcoreMesh(core_axis_name="c", subcore_axis_name="s")


def _widen_body(t_ref, o_ref):
    x = t_ref[...]
    o_ref[...] = jnp.concatenate([x, x], axis=1)


_WROWS = 8000

_widen_tc = pl.pallas_call(
    _widen_body,
    out_shape=jax.ShapeDtypeStruct((VOCAB_ROWS, 2 * EMBED), jnp.float32),
    grid=(VOCAB_ROWS // _WROWS,),
    in_specs=[pl.BlockSpec((_WROWS, EMBED), lambda i: (i, 0))],
    out_specs=pl.BlockSpec((_WROWS, 2 * EMBED), lambda i: (i, 0)),
)


@functools.partial(
    pl.kernel,
    out_type=jax.ShapeDtypeStruct((BATCH * EMBED,), jnp.float32),
    mesh=_mesh,
    compiler_params=pltpu.CompilerParams(use_tc_tiling_on_sc=True),
    scratch_types=[
        pltpu.VMEM((SPT * HIST,), jnp.int32),
        pltpu.VMEM((NBUF, HIST, 2 * EMBED), jnp.float32),
        pltpu.VMEM((SPT * EMBED,), jnp.float32),
        [pltpu.SemaphoreType.DMA] * NBUF,
    ],
)
def _pool_sc(x_hbm, table_hbm, out_hbm, idx_v, rows_v, pooled_v, sems):
    wid = lax.axis_index("s") * NC + lax.axis_index("c")
    pltpu.sync_copy(x_hbm.at[pl.ds(wid * (SPT * HIST), SPT * HIST)], idx_v)

    def issue(s, b):
        off = pl.multiple_of(s * HIST, 8)
        pltpu.async_copy(table_hbm.at[idx_v.at[pl.ds(off, C0)]],
                         rows_v.at[b, pl.ds(0, C0)], sems[b])
        pltpu.async_copy(table_hbm.at[idx_v.at[pl.ds(off + C0, C1)]],
                         rows_v.at[b, pl.ds(C0, C1)], sems[b])

    def drain(b):
        pltpu.make_async_copy(table_hbm.at[idx_v.at[pl.ds(0, C0)]],
                              rows_v.at[b, pl.ds(0, C0)], sems[b]).wait()
        pltpu.make_async_copy(table_hbm.at[idx_v.at[pl.ds(0, C1)]],
                              rows_v.at[b, pl.ds(C0, C1)], sems[b]).wait()

    for b in range(NBUF):
        issue(b, b)

    def accum(s, b):
        drain(b)

        def body(r, acc):
            return tuple(acc[j] + rows_v[b, r, pl.ds(16 * j, 16)]
                         for j in range(VR))

        z = jnp.zeros((16,), jnp.float32)
        acc = lax.fori_loop(0, HIST, body, (z,) * VR, unroll=8)
        for j in range(VR):
            pooled_v[pl.ds(s * EMBED + 16 * j, 16)] = acc[j]

    NFULL = SPT // NBUF

    def group(i, carry):
        sb = i * NBUF
        for b in range(NBUF):
            s = sb + b
            accum(s, b)

            @pl.when(s + NBUF < SPT)
            def _():
                issue(s + NBUF, b)
        return carry

    lax.fori_loop(0, NFULL, group, 0)
    for t in range(SPT % NBUF):
        accum(NFULL * NBUF + t, t)
    pltpu.sync_copy(pooled_v,
                    out_hbm.at[pl.ds(wid * (SPT * EMBED), SPT * EMBED)])


def _head_body(ps_ref, w_ref, b_ref, o_ref):
    pooled = ps_ref[...] * (1.0 / HIST)
    out = lax.dot_general(pooled, w_ref[...], (((1,), (1,)), ((), ())),
                          preferred_element_type=jnp.float32)
    out = out + b_ref[...]
    ss = jnp.sum(out * out, axis=1, keepdims=True)
    o_ref[...] = out / jnp.maximum(jnp.sqrt(ss), 1e-12)


_head_tc = pl.pallas_call(
    _head_body,
    out_shape=jax.ShapeDtypeStruct((BATCH, OUT_DIM), jnp.float32),
    grid=(4,),
    in_specs=[
        pl.BlockSpec((BATCH // 4, EMBED), lambda i: (i, 0)),
        pl.BlockSpec((OUT_DIM, EMBED), lambda i: (0, 0)),
        pl.BlockSpec((1, OUT_DIM), lambda i: (0, 0)),
    ],
    out_specs=pl.BlockSpec((BATCH // 4, OUT_DIM), lambda i: (i, 0)),
)


def kernel(x, table, W, b):
    xf = x.astype(jnp.int32).reshape(-1)
    t2 = _widen_tc(table)
    sums = _pool_sc(xf, t2).reshape(BATCH, EMBED)
    return _head_tc(sums, W, b.reshape(1, OUT_DIM))


# speedup vs baseline: 1.1833x; 1.1833x over previous
import functools

import jax
import jax.numpy as jnp
from jax import lax
from jax.experimental import pallas as pl
from jax.experimental.pallas import tpu as pltpu
from jax.experimental.pallas import tpu_sc as plsc

VOCAB_ROWS = 1000000
EMBED = 64
OUT_DIM = 128
BATCH = 4096
HIST = 200

NC = 2
NS = 16
NW = NC * NS
SPT = BATCH // NW
C0, C1 = 104, 96
VR = EMBED // 16
NBUF = 3

_mesh = plsc.VectorSubcoreMesh(core_axis_name="c", subcore_axis_name="s")


def _widen_body(t_ref, o_ref):
    x = t_ref[...].reshape(_WROWS, EMBED)
    o_ref[...] = jnp.concatenate([x, x], axis=1)


_WROWS = 8000

_widen_tc = pl.pallas_call(
    _widen_body,
    out_shape=jax.ShapeDtypeStruct((VOCAB_ROWS, 2 * EMBED), jnp.float32),
    grid=(VOCAB_ROWS // _WROWS,),
    in_specs=[pl.BlockSpec((_WROWS // 8, 8, EMBED), lambda i: (i, 0, 0))],
    out_specs=pl.BlockSpec((_WROWS, 2 * EMBED), lambda i: (i, 0)),
)


@functools.partial(
    pl.kernel,
    out_type=jax.ShapeDtypeStruct((BATCH * EMBED,), jnp.float32),
    mesh=_mesh,
    compiler_params=pltpu.CompilerParams(use_tc_tiling_on_sc=True),
    scratch_types=[
        pltpu.VMEM((SPT * HIST,), jnp.int32),
        pltpu.VMEM((NBUF, HIST, 2 * EMBED), jnp.float32),
        pltpu.VMEM((SPT * EMBED,), jnp.float32),
        [pltpu.SemaphoreType.DMA] * NBUF,
    ],
)
def _pool_sc(x_hbm, table_hbm, out_hbm, idx_v, rows_v, pooled_v, sems):
    wid = lax.axis_index("s") * NC + lax.axis_index("c")
    pltpu.sync_copy(x_hbm.at[pl.ds(wid * (SPT * HIST), SPT * HIST)], idx_v)

    def issue(s, b):
        off = pl.multiple_of(s * HIST, 8)
        pltpu.async_copy(table_hbm.at[idx_v.at[pl.ds(off, C0)]],
                         rows_v.at[b, pl.ds(0, C0)], sems[b])
        pltpu.async_copy(table_hbm.at[idx_v.at[pl.ds(off + C0, C1)]],
                         rows_v.at[b, pl.ds(C0, C1)], sems[b])

    def drain(b):
        pltpu.make_async_copy(table_hbm.at[idx_v.at[pl.ds(0, C0)]],
                              rows_v.at[b, pl.ds(0, C0)], sems[b]).wait()
        pltpu.make_async_copy(table_hbm.at[idx_v.at[pl.ds(0, C1)]],
                              rows_v.at[b, pl.ds(C0, C1)], sems[b]).wait()

    for b in range(NBUF):
        issue(b, b)

    def accum(s, b):
        drain(b)

        def body(r, acc):
            return tuple(acc[j] + rows_v[b, r, pl.ds(16 * j, 16)]
                         for j in range(VR))

        z = jnp.zeros((16,), jnp.float32)
        acc = lax.fori_loop(0, HIST, body, (z,) * VR, unroll=8)
        for j in range(VR):
            pooled_v[pl.ds(s * EMBED + 16 * j, 16)] = acc[j]

    NFULL = SPT // NBUF

    def group(i, carry):
        sb = i * NBUF
        for b in range(NBUF):
            s = sb + b
            accum(s, b)

            @pl.when(s + NBUF < SPT)
            def _():
                issue(s + NBUF, b)
        return carry

    lax.fori_loop(0, NFULL, group, 0)
    for t in range(SPT % NBUF):
        accum(NFULL * NBUF + t, t)
    pltpu.sync_copy(pooled_v,
                    out_hbm.at[pl.ds(wid * (SPT * EMBED), SPT * EMBED)])


def _head_body(ps_ref, w_ref, b_ref, o_ref):
    pooled = ps_ref[...] * (1.0 / HIST)
    out = lax.dot_general(pooled, w_ref[...], (((1,), (1,)), ((), ())),
                          preferred_element_type=jnp.float32)
    out = out + b_ref[...]
    ss = jnp.sum(out * out, axis=1, keepdims=True)
    o_ref[...] = out / jnp.maximum(jnp.sqrt(ss), 1e-12)


_head_tc = pl.pallas_call(
    _head_body,
    out_shape=jax.ShapeDtypeStruct((BATCH, OUT_DIM), jnp.float32),
    grid=(4,),
    in_specs=[
        pl.BlockSpec((BATCH // 4, EMBED), lambda i: (i, 0)),
        pl.BlockSpec((OUT_DIM, EMBED), lambda i: (0, 0)),
        pl.BlockSpec((1, OUT_DIM), lambda i: (0, 0)),
    ],
    out_specs=pl.BlockSpec((BATCH // 4, OUT_DIM), lambda i: (i, 0)),
)


def kernel(x, table, W, b):
    xf = x.astype(jnp.int32).reshape(-1)
    t2 = _widen_tc(table.reshape(VOCAB_ROWS // 8, 8, EMBED))
    sums = _pool_sc(xf, t2).reshape(BATCH, EMBED)
    return _head_tc(sums, W, b.reshape(1, OUT_DIM))
